# Initial kernel scaffold; baseline (speedup 1.0000x reference)
import jax
import jax.numpy as jnp
from jax.experimental import pallas as pl


def kernel(x, edge_index, batch_index, W0, b0, W1, b1, W2, b2, W_out, b_out):
    raise NotImplementedError("write your pallas kernel here")


# speedup vs baseline: 13.3364x; 13.3364x over previous
import functools

import jax
import jax.numpy as jnp
from jax import lax
from jax.experimental import pallas as pl
from jax.experimental.pallas import tpu as pltpu
from jax.experimental.pallas import tpu_sc as plsc

N = 10000
E = 320000
F_IN = 128
H = 64
B = 64

NC = 2
NS = 16
NW = NC * NS
CH = 80
EPW = E // NW
NP = 10240
NPW = NP // NS


def _deg_body(dst_hbm, ones_hbm, zeros_hbm, out_hbm, didx, ones_v, acc, sem):
    c = lax.axis_index("c")
    s = lax.axis_index("s")
    pltpu.sync_copy(zeros_hbm.at[pl.ds(s * NPW, NPW)], acc.at[pl.ds(s * NPW, NPW)])
    pltpu.sync_copy(ones_hbm, ones_v)
    plsc.subcore_barrier()
    base = (c * NS + s) * EPW

    def body(i, carry):
        off = pl.multiple_of(base + i * CH, 8)
        pltpu.sync_copy(dst_hbm.at[pl.ds(off, CH)], didx)
        pltpu.sync_copy(ones_v, acc.at[didx], add=True)
        return carry

    lax.fori_loop(0, EPW // CH, body, 0)
    plsc.subcore_barrier()
    pltpu.sync_copy(acc.at[pl.ds(s * NPW, NPW)],
                    out_hbm.at[c].at[pl.ds(s * NPW, NPW)])


def _agg_body(hs_hbm, src_hbm, dst_hbm, zeros_hbm, out_hbm,
              sidx, didx, rows, hs_s, acc, sem):
    c = lax.axis_index("c")
    s = lax.axis_index("s")
    pltpu.sync_copy(zeros_hbm.at[pl.ds(s * NPW, NPW)], acc.at[pl.ds(s * NPW, NPW)])
    pltpu.sync_copy(hs_hbm.at[pl.ds(s * NPW, NPW)], hs_s.at[pl.ds(s * NPW, NPW)])
    plsc.subcore_barrier()
    base = (c * NS + s) * EPW

    def body(i, carry):
        off = pl.multiple_of(base + i * CH, 8)
        pltpu.sync_copy(src_hbm.at[pl.ds(off, CH)], sidx)
        pltpu.sync_copy(dst_hbm.at[pl.ds(off, CH)], didx)
        pltpu.async_copy(hs_s.at[sidx], rows, sem).wait()
        pltpu.sync_copy(rows, acc.at[didx], add=True)
        return carry

    lax.fori_loop(0, EPW // CH, body, 0)
    plsc.subcore_barrier()
    pltpu.sync_copy(acc.at[pl.ds(s * NPW, NPW)],
                    out_hbm.at[c].at[pl.ds(s * NPW, NPW)])


@functools.cache
def _sc_calls():
    mesh = plsc.VectorSubcoreMesh(
        core_axis_name="c", subcore_axis_name="s",
        num_cores=NC, num_subcores=NS)
    params = pltpu.CompilerParams(use_tc_tiling_on_sc=False)
    deg_call = pl.kernel(
        _deg_body,
        out_type=jax.ShapeDtypeStruct((NC, NP, 16), jnp.float32),
        mesh=mesh,
        compiler_params=params,
        scratch_types=[
            pltpu.VMEM((CH,), jnp.int32),
            pltpu.VMEM((CH, 16), jnp.float32),
            pltpu.VMEM_SHARED((NP, 16), jnp.float32),
            pltpu.SemaphoreType.DMA,
        ],
    )
    agg_call = pl.kernel(
        _agg_body,
        out_type=jax.ShapeDtypeStruct((NC, NP, H), jnp.float32),
        mesh=mesh,
        compiler_params=params,
        scratch_types=[
            pltpu.VMEM((CH,), jnp.int32),
            pltpu.VMEM((CH,), jnp.int32),
            pltpu.VMEM((CH, H), jnp.float32),
            pltpu.VMEM_SHARED((NP, H), jnp.float32),
            pltpu.VMEM_SHARED((NP, H), jnp.float32),
            pltpu.SemaphoreType.DMA,
        ],
    )
    return deg_call, agg_call


def _dot(a, b):
    return lax.dot_general(a, b, (((1,), (0,)), ((), ())),
                           preferred_element_type=jnp.float32)


def _prep_body(x_ref, dega_ref, degb_ref, w_ref, dinv_ref, hs_ref):
    deg = dega_ref[0:N, 0:1] + degb_ref[0:N, 0:1] + 1.0
    dinv = lax.rsqrt(deg)
    dinv_ref[...] = dinv
    hs_ref[0:N, :] = dinv * _dot(x_ref[...], w_ref[...])


def _update_body(acc_ref, hs_ref, dinv_ref, b_ref, w_ref, out_ref):
    dinv = dinv_ref[...]
    t = jnp.tanh(dinv * (acc_ref[0, 0:N, :] + acc_ref[1, 0:N, :]
                         + hs_ref[0:N, :]) + b_ref[...])
    out_ref[0:N, :] = dinv * _dot(t, w_ref[...])


def _final_body(acc_ref, hs_ref, dinv_ref, b_ref, batch_ref,
                wout_ref, bout_ref, out_ref, t_ref, gmax_ref, gsum_ref, cnt_ref):
    dinv = dinv_ref[...]
    t_ref[...] = jnp.tanh(
        dinv * (acc_ref[0, 0:N, :] + acc_ref[1, 0:N, :]
                + hs_ref[0:N, :]) + b_ref[...])

    def body(b, carry):
        t = t_ref[...]
        mask = batch_ref[...] == b
        gmax_ref[pl.ds(b, 1), :] = jnp.max(
            jnp.where(mask, t, -jnp.inf), axis=0, keepdims=True)
        gsum_ref[pl.ds(b, 1), :] = jnp.sum(
            jnp.where(mask, t, 0.0), axis=0, keepdims=True)
        cnt_ref[pl.ds(b, 1), :] = jnp.sum(
            mask.astype(jnp.float32), axis=0, keepdims=True)
        return carry

    lax.fori_loop(0, B, body, 0)
    gmean = gsum_ref[...] / jnp.maximum(cnt_ref[...], 1.0)
    pooled = jnp.concatenate([gmax_ref[...], gmean], axis=1)
    out_ref[...] = _dot(pooled, wout_ref[...]) + bout_ref[...]


_prep_call = pl.pallas_call(
    _prep_body,
    out_shape=[jax.ShapeDtypeStruct((N, 1), jnp.float32),
               jax.ShapeDtypeStruct((NP, H), jnp.float32)],
)

_update_call = pl.pallas_call(
    _update_body,
    out_shape=jax.ShapeDtypeStruct((NP, H), jnp.float32),
)

_final_call = pl.pallas_call(
    _final_body,
    out_shape=jax.ShapeDtypeStruct((B, 1), jnp.float32),
    scratch_shapes=[
        pltpu.VMEM((N, H), jnp.float32),
        pltpu.VMEM((B, H), jnp.float32),
        pltpu.VMEM((B, H), jnp.float32),
        pltpu.VMEM((B, 1), jnp.float32),
    ],
)


def kernel(x, edge_index, batch_index, W0, b0, W1, b1, W2, b2, W_out, b_out):
    src = edge_index[0]
    dst = edge_index[1]
    zeros_h = jnp.zeros((NP, H), jnp.float32)
    zeros_16 = jnp.zeros((NP, 16), jnp.float32)
    ones_16 = jnp.ones((CH, 16), jnp.float32)
    batch2d = batch_index.reshape(N, 1)

    deg_call, agg_call = _sc_calls()

    deg = deg_call(dst, ones_16, zeros_16)
    dinv, hs = _prep_call(x, deg[0], deg[1], W0)

    acc = agg_call(hs, src, dst, zeros_h)
    hs = _update_call(acc, hs, dinv, b0.reshape(1, H), W1)

    acc = agg_call(hs, src, dst, zeros_h)
    hs = _update_call(acc, hs, dinv, b1.reshape(1, H), W2)

    acc = agg_call(hs, src, dst, zeros_h)
    out = _final_call(acc, hs, dinv, b2.reshape(1, H), batch2d,
                      W_out, b_out.reshape(1, 1))
    return out


# speedup vs baseline: 24.2156x; 1.8157x over previous
import functools

import jax
import jax.numpy as jnp
from jax import lax
from jax.experimental import pallas as pl
from jax.experimental.pallas import tpu as pltpu
from jax.experimental.pallas import tpu_sc as plsc

N = 10000
E = 320000
F_IN = 128
H = 64
B = 64

NC = 2
NS = 16
NW = NC * NS
CH = 80
EPW = E // NW
NP = 10240
NPW = NP // NS
NIT = EPW // CH


def _deg_body(eidx_hbm, ones_hbm, zeros_hbm, out_hbm, didx_all, ones_v, acc, sem):
    c = lax.axis_index("c")
    s = lax.axis_index("s")
    w = c * NS + s
    pltpu.sync_copy(zeros_hbm.at[pl.ds(s * NPW, NPW)], acc.at[pl.ds(s * NPW, NPW)])
    pltpu.sync_copy(eidx_hbm.at[1].at[pl.ds(w * NIT, NIT)], didx_all)
    pltpu.sync_copy(ones_hbm, ones_v)
    plsc.subcore_barrier()

    def body(i, carry):
        pltpu.async_copy(ones_v, acc.at[didx_all.at[i]], sem, add=True)
        return carry

    lax.fori_loop(0, NIT, body, 0)

    def drain(i, carry):
        pltpu.make_async_copy(ones_v, acc.at[didx_all.at[i]], sem).wait()
        return carry

    lax.fori_loop(0, NIT, drain, 0)
    plsc.subcore_barrier()
    pltpu.sync_copy(acc.at[pl.ds(s * NPW, NPW)],
                    out_hbm.at[c].at[pl.ds(s * NPW, NPW)])


def _agg_body(hs_hbm, eidx_hbm, zeros_hbm, out_hbm,
              sidx_all, didx_all, rows0, rows1, hs_s, acc, sem0, sem1):
    c = lax.axis_index("c")
    s = lax.axis_index("s")
    w = c * NS + s
    pltpu.sync_copy(zeros_hbm.at[pl.ds(s * NPW, NPW)], acc.at[pl.ds(s * NPW, NPW)])
    pltpu.sync_copy(hs_hbm.at[pl.ds(s * NPW, NPW)], hs_s.at[pl.ds(s * NPW, NPW)])
    pltpu.sync_copy(eidx_hbm.at[0].at[pl.ds(w * NIT, NIT)], sidx_all)
    pltpu.sync_copy(eidx_hbm.at[1].at[pl.ds(w * NIT, NIT)], didx_all)
    plsc.subcore_barrier()

    def wait_gather(buf, sem):
        pltpu.make_async_copy(hs_s.at[sidx_all.at[0]], buf, sem).wait()

    pltpu.async_copy(hs_s.at[sidx_all.at[0]], rows0, sem0)

    def body(i, carry):
        even = lax.rem(i, 2) == 0

        @pl.when(even)
        def _():
            wait_gather(rows0, sem0)

            @pl.when(i + 1 < NIT)
            def _():
                pltpu.async_copy(hs_s.at[sidx_all.at[i + 1]], rows1, sem1)

            pltpu.sync_copy(rows0, acc.at[didx_all.at[i]], add=True)

        @pl.when(jnp.logical_not(even))
        def _():
            wait_gather(rows1, sem1)

            @pl.when(i + 1 < NIT)
            def _():
                pltpu.async_copy(hs_s.at[sidx_all.at[i + 1]], rows0, sem0)

            pltpu.sync_copy(rows1, acc.at[didx_all.at[i]], add=True)

        return carry

    lax.fori_loop(0, NIT, body, 0)
    plsc.subcore_barrier()
    pltpu.sync_copy(acc.at[pl.ds(s * NPW, NPW)],
                    out_hbm.at[c].at[pl.ds(s * NPW, NPW)])


@functools.cache
def _sc_calls():
    mesh = plsc.VectorSubcoreMesh(
        core_axis_name="c", subcore_axis_name="s",
        num_cores=NC, num_subcores=NS)
    params = pltpu.CompilerParams(use_tc_tiling_on_sc=False)
    deg_call = pl.kernel(
        _deg_body,
        out_type=jax.ShapeDtypeStruct((NC, NP, 16), jnp.float32),
        mesh=mesh,
        compiler_params=params,
        scratch_types=[
            pltpu.VMEM((NIT, CH), jnp.int32),
            pltpu.VMEM((CH, 16), jnp.float32),
            pltpu.VMEM_SHARED((NP, 16), jnp.float32),
            pltpu.SemaphoreType.DMA,
        ],
    )
    agg_call = pl.kernel(
        _agg_body,
        out_type=jax.ShapeDtypeStruct((NC, NP, H), jnp.float32),
        mesh=mesh,
        compiler_params=params,
        scratch_types=[
            pltpu.VMEM((NIT, CH), jnp.int32),
            pltpu.VMEM((NIT, CH), jnp.int32),
            pltpu.VMEM((CH, H), jnp.float32),
            pltpu.VMEM((CH, H), jnp.float32),
            pltpu.VMEM_SHARED((NP, H), jnp.float32),
            pltpu.VMEM_SHARED((NP, H), jnp.float32),
            pltpu.SemaphoreType.DMA,
            pltpu.SemaphoreType.DMA,
        ],
    )
    return deg_call, agg_call


def _dot(a, b):
    return lax.dot_general(a, b, (((1,), (0,)), ((), ())),
                           preferred_element_type=jnp.float32)


def _prep_body(x_ref, dega_ref, degb_ref, w_ref, dinv_ref, hs_ref):
    deg = dega_ref[0:N, 0:1] + degb_ref[0:N, 0:1] + 1.0
    dinv = lax.rsqrt(deg)
    dinv_ref[...] = dinv
    hs_ref[0:N, :] = dinv * _dot(x_ref[...], w_ref[...])


def _update_body(acc_ref, hs_ref, dinv_ref, b_ref, w_ref, out_ref):
    dinv = dinv_ref[...]
    t = jnp.tanh(dinv * (acc_ref[0, 0:N, :] + acc_ref[1, 0:N, :]
                         + hs_ref[0:N, :]) + b_ref[...])
    out_ref[0:N, :] = dinv * _dot(t, w_ref[...])


def _final_body(acc_ref, hs_ref, dinv_ref, b_ref, batch_ref,
                wout_ref, bout_ref, out_ref, t_ref, gmax_ref, gsum_ref, cnt_ref):
    dinv = dinv_ref[...]
    t_ref[...] = jnp.tanh(
        dinv * (acc_ref[0, 0:N, :] + acc_ref[1, 0:N, :]
                + hs_ref[0:N, :]) + b_ref[...])

    def body(b, carry):
        t = t_ref[...]
        mask = batch_ref[...] == b
        gmax_ref[pl.ds(b, 1), :] = jnp.max(
            jnp.where(mask, t, -jnp.inf), axis=0, keepdims=True)
        gsum_ref[pl.ds(b, 1), :] = jnp.sum(
            jnp.where(mask, t, 0.0), axis=0, keepdims=True)
        cnt_ref[pl.ds(b, 1), :] = jnp.sum(
            mask.astype(jnp.float32), axis=0, keepdims=True)
        return carry

    lax.fori_loop(0, B, body, 0)
    gmean = gsum_ref[...] / jnp.maximum(cnt_ref[...], 1.0)
    pooled = jnp.concatenate([gmax_ref[...], gmean], axis=1)
    out_ref[...] = _dot(pooled, wout_ref[...]) + bout_ref[...]


_prep_call = pl.pallas_call(
    _prep_body,
    out_shape=[jax.ShapeDtypeStruct((N, 1), jnp.float32),
               jax.ShapeDtypeStruct((NP, H), jnp.float32)],
)

_update_call = pl.pallas_call(
    _update_body,
    out_shape=jax.ShapeDtypeStruct((NP, H), jnp.float32),
)

_final_call = pl.pallas_call(
    _final_body,
    out_shape=jax.ShapeDtypeStruct((B, 1), jnp.float32),
    scratch_shapes=[
        pltpu.VMEM((N, H), jnp.float32),
        pltpu.VMEM((B, H), jnp.float32),
        pltpu.VMEM((B, H), jnp.float32),
        pltpu.VMEM((B, 1), jnp.float32),
    ],
)


def kernel(x, edge_index, batch_index, W0, b0, W1, b1, W2, b2, W_out, b_out):
    eidx = edge_index.reshape(2, E // CH, CH)
    zeros_h = jnp.zeros((NP, H), jnp.float32)
    zeros_16 = jnp.zeros((NP, 16), jnp.float32)
    ones_16 = jnp.ones((CH, 16), jnp.float32)
    batch2d = batch_index.reshape(N, 1)

    deg_call, agg_call = _sc_calls()

    deg = deg_call(eidx, ones_16, zeros_16)
    dinv, hs = _prep_call(x, deg[0], deg[1], W0)

    acc = agg_call(hs, eidx, zeros_h)
    hs = _update_call(acc, hs, dinv, b0.reshape(1, H), W1)

    acc = agg_call(hs, eidx, zeros_h)
    hs = _update_call(acc, hs, dinv, b1.reshape(1, H), W2)

    acc = agg_call(hs, eidx, zeros_h)
    out = _final_call(acc, hs, dinv, b2.reshape(1, H), batch2d,
                      W_out, b_out.reshape(1, 1))
    return out
